# baseline (speedup 1.0000x reference)
import jax
import jax.numpy as jnp
from jax.experimental import pallas as pl
from jax.experimental.pallas import tpu as pltpu

_N = 16384
_NBINS = 50
_NBLK = 26
_SYNC_COST = 10.0
_TARGET = 0.1


def _body(x_ref, out_ref, pend_ref, max_ref):
    k = pl.program_id(0)
    x = x_ref[0]
    part_a = jnp.sum(x[0:10], axis=0, keepdims=True)
    mid = jnp.sum(x[10:30], axis=0, keepdims=True)
    part_b = jnp.sum(x[30:40], axis=0, keepdims=True)

    @pl.when(k == 0)
    def _():
        max_ref[0] = 0.0

    @pl.when(k > 0)
    def _():
        tot = pend_ref[...] + part_a
        cnt = jnp.sum((tot != 0.0).astype(jnp.float32))
        max_ref[0] = jnp.maximum(max_ref[0], cnt)

    @pl.when(k < _NBLK - 1)
    def _():
        cnt_mid = jnp.sum((mid != 0.0).astype(jnp.float32))
        max_ref[0] = jnp.maximum(max_ref[0], cnt_mid)
        pend_ref[...] = part_b

    @pl.when(k == _NBLK - 1)
    def _():
        frac = max_ref[0] / jnp.float32(_N)
        d = frac - jnp.float32(_TARGET)
        out_ref[0, 0] = jnp.float32(_SYNC_COST) * d * d


def kernel(spikes):
    out = pl.pallas_call(
        _body,
        grid=(_NBLK,),
        in_specs=[
            pl.BlockSpec((1, 40, _N), lambda k: (0, k + 1, 0))
        ],
        out_specs=pl.BlockSpec(memory_space=pltpu.SMEM),
        out_shape=jax.ShapeDtypeStruct((1, 1), jnp.float32),
        scratch_shapes=[
            pltpu.VMEM((1, _N), jnp.float32),
            pltpu.SMEM((1,), jnp.float32),
        ],
    )(spikes)
    return out[0, 0]


# speedup vs baseline: 1.0152x; 1.0152x over previous
import jax
import jax.numpy as jnp
from jax.experimental import pallas as pl
from jax.experimental.pallas import tpu as pltpu

_N = 16384
_C = 8
_NC = _N // _C
_NBINS = 50
_NBLK = 26
_SYNC_COST = 10.0
_TARGET = 0.1


def _body(*refs):
    x_refs = refs[:_C]
    out_ref = refs[_C]
    pend_ref, max_ref = refs[_C + 1], refs[_C + 2]
    k = pl.program_id(0)

    @pl.when(k == 0)
    def _():
        max_ref[0] = 0.0

    cnt_old = jnp.float32(0.0)
    cnt_mid = jnp.float32(0.0)
    for c in range(_C):
        x = x_refs[c][0]
        part_a = jnp.sum(x[0:10], axis=0, keepdims=True)
        mid = jnp.sum(x[10:30], axis=0, keepdims=True)
        part_b = jnp.sum(x[30:40], axis=0, keepdims=True)
        tot = pend_ref[:, c * _NC:(c + 1) * _NC] + part_a
        cnt_old = cnt_old + jnp.sum((tot != 0.0).astype(jnp.float32))
        cnt_mid = cnt_mid + jnp.sum((mid != 0.0).astype(jnp.float32))
        pend_ref[:, c * _NC:(c + 1) * _NC] = part_b

    @pl.when(k > 0)
    def _():
        max_ref[0] = jnp.maximum(max_ref[0], cnt_old)

    @pl.when(k < _NBLK - 1)
    def _():
        max_ref[0] = jnp.maximum(max_ref[0], cnt_mid)

    @pl.when(k == _NBLK - 1)
    def _():
        frac = max_ref[0] / jnp.float32(_N)
        d = frac - jnp.float32(_TARGET)
        out_ref[0, 0] = jnp.float32(_SYNC_COST) * d * d


def kernel(spikes):
    def _mk_spec(c):
        return pl.BlockSpec((1, 40, _NC), lambda k, c=c: (0, k + 1, c))

    out = pl.pallas_call(
        _body,
        grid=(_NBLK,),
        in_specs=[_mk_spec(c) for c in range(_C)],
        out_specs=pl.BlockSpec(memory_space=pltpu.SMEM),
        out_shape=jax.ShapeDtypeStruct((1, 1), jnp.float32),
        scratch_shapes=[
            pltpu.VMEM((1, _N), jnp.float32),
            pltpu.SMEM((1,), jnp.float32),
        ],
    )(*([spikes] * _C))
    return out[0, 0]
